# Initial kernel scaffold; baseline (speedup 1.0000x reference)
import jax
import jax.numpy as jnp
from jax.experimental import pallas as pl


def kernel(hidden_states, router_weight, gate_up_proj, down_proj, shared_gate_proj, shared_up_proj, shared_down_proj, shared_expert_gate_weight):
    raise NotImplementedError("write your pallas kernel here")


# speedup vs baseline: 1.0056x; 1.0056x over previous
import functools

import jax
import jax.numpy as jnp
from jax.experimental import pallas as pl
from jax.experimental.pallas import tpu as pltpu

NUM_EXPERTS = 8
TOP_K = 2
HIDDEN = 1024
FF = 1408
TOKENS = 2048
TB = 512
NEG = -1e30


def _moe_body(x_ref, rwp_ref, gu_ref, dn_ref, out_ref, scr_ref):
    e = pl.program_id(0)

    @pl.when(e == 0)
    def _router():
        logits = jnp.dot(x_ref[...], rwp_ref[...],
                         preferred_element_type=jnp.float32)
        lane = jax.lax.broadcasted_iota(jnp.int32, logits.shape, 1)
        l = jnp.where(lane < NUM_EXPERTS, logits, NEG)
        m0 = jnp.max(l, axis=1, keepdims=True)
        e0 = jnp.min(jnp.where(l >= m0, lane, 9999), axis=1, keepdims=True)
        l2 = jnp.where(lane == e0, NEG, l)
        m1 = jnp.max(l2, axis=1, keepdims=True)
        e1 = jnp.min(jnp.where(l2 >= m1, lane, 9999), axis=1, keepdims=True)
        w0 = 1.0 / (1.0 + jnp.exp(m1 - m0))
        scr_ref[:, 0:1] = w0
        scr_ref[:, 1:2] = 1.0 - w0
        scr_ref[:, 2:3] = e0.astype(jnp.float32)
        scr_ref[:, 3:4] = e1.astype(jnp.float32)
        g = logits[:, NUM_EXPERTS:NUM_EXPERTS + 1]
        scr_ref[:, 4:5] = 1.0 / (1.0 + jnp.exp(-g))
        out_ref[...] = jnp.zeros_like(out_ref)

    ef = e.astype(jnp.float32)
    gu_w = gu_ref[0]
    dn_w = dn_ref[0]
    for tb in range(TOKENS // TB):
        sl = pl.ds(tb * TB, TB)
        xb = x_ref[sl, :].astype(jnp.bfloat16)
        gu = jnp.dot(xb, gu_w, preferred_element_type=jnp.float32)
        h = (gu[:, :FF] * (1.0 / (1.0 + jnp.exp(-gu[:, :FF])))
             * gu[:, FF:]).astype(jnp.bfloat16)
        eo = jnp.dot(h, dn_w, preferred_element_type=jnp.float32)
        w0 = scr_ref[sl, 0:1]
        w1 = scr_ref[sl, 1:2]
        e0 = scr_ref[sl, 2:3]
        e1 = scr_ref[sl, 3:4]
        gate = scr_ref[sl, 4:5]
        we = jnp.where(e == NUM_EXPERTS, gate,
                       jnp.where(e0 == ef, w0, 0.0)
                       + jnp.where(e1 == ef, w1, 0.0))
        out_ref[sl, :] += we * eo


@jax.jit
def _run(x, rwp_t, gu_t, dn_t):
    f = pl.pallas_call(
        _moe_body,
        grid=(NUM_EXPERTS + 1,),
        in_specs=[
            pl.BlockSpec((TOKENS, HIDDEN), lambda e: (0, 0)),
            pl.BlockSpec((HIDDEN, 128), lambda e: (0, 0)),
            pl.BlockSpec((1, HIDDEN, 2 * FF), lambda e: (e, 0, 0)),
            pl.BlockSpec((1, FF, HIDDEN), lambda e: (e, 0, 0)),
        ],
        out_specs=pl.BlockSpec((TOKENS, HIDDEN), lambda e: (0, 0)),
        out_shape=jax.ShapeDtypeStruct((TOKENS, HIDDEN), jnp.float32),
        scratch_shapes=[pltpu.VMEM((TOKENS, 128), jnp.float32)],
        compiler_params=pltpu.CompilerParams(
            dimension_semantics=("arbitrary",)),
    )
    return f(x, rwp_t, gu_t, dn_t)


def kernel(hidden_states, router_weight, gate_up_proj, down_proj,
           shared_gate_proj, shared_up_proj, shared_down_proj,
           shared_expert_gate_weight):
    B, S, H = hidden_states.shape
    x = hidden_states.reshape(-1, H)
    rwp = jnp.zeros((128, H), jnp.float32)
    rwp = rwp.at[:NUM_EXPERTS].set(router_weight)
    rwp = rwp.at[NUM_EXPERTS].set(shared_expert_gate_weight[0])
    sgu = jnp.concatenate([shared_gate_proj, shared_up_proj], axis=0)
    gu_t = jnp.concatenate(
        [gate_up_proj, sgu[None]], axis=0).transpose(0, 2, 1)
    dn_t = jnp.concatenate(
        [down_proj, shared_down_proj[None]], axis=0).transpose(0, 2, 1)
    out = _run(x, rwp.T,
               gu_t.astype(jnp.bfloat16), dn_t.astype(jnp.bfloat16))
    return out.reshape(B, S, H)
